# baseline (speedup 1.0000x reference)
import jax
import jax.numpy as jnp
from jax.experimental import pallas as pl

_H, _W = 50, 38
_PW = _W + 2
_N = 2048
_C = 512


def _body(f_ref, w_ref, b1_ref, cw_ref, cb_ref, out_ref):
    acc = jnp.zeros((_C, _N), jnp.float32)
    for ky in range(3):
        for kx in range(3):
            t = ky * 3 + kx
            off = ky * _PW + kx
            acc = acc + jnp.dot(w_ref[t], f_ref[:, off:off + _N],
                                preferred_element_type=jnp.float32)
    acc = acc + b1_ref[:]
    out_ref[:] = jnp.dot(cw_ref[:], acc,
                         preferred_element_type=jnp.float32) + cb_ref[:]


def kernel(out_map, conv1_w, conv1_b, loc_w, loc_b, score_w, score_b):
    x = out_map[0]
    xp = jnp.pad(x, ((0, 0), (1, 1), (1, 1)))
    f = xp.reshape(_C, (_H + 2) * _PW)
    f = jnp.pad(f, ((0, 0), (0, _N + 2 * _PW + 2 - f.shape[1])))
    wt = conv1_w.transpose(2, 3, 0, 1).reshape(9, _C, _C)
    cw = jnp.concatenate([loc_w[:, :, 0, 0], score_w[:, :, 0, 0]], axis=0)
    cw = jnp.pad(cw, ((0, 64 - cw.shape[0]), (0, 0)))
    cb = jnp.pad(jnp.concatenate([loc_b, score_b]), (0, 10)).reshape(64, 1)
    b1 = conv1_b.reshape(_C, 1)

    out = pl.pallas_call(
        _body,
        out_shape=jax.ShapeDtypeStruct((64, _N), jnp.float32),
    )(f, wt, b1, cw, cb)

    out = out[:, :_H * _PW].reshape(64, _H, _PW)[:, :, :_W]
    loc = out[:36][None]
    score = out[36:54][None]
    return (loc, score)
